# baseline (speedup 1.0000x reference)
import functools

import jax
import jax.numpy as jnp
from jax import lax
from jax.experimental import pallas as pl
from jax.experimental.pallas import tpu as pltpu
from jax.experimental.pallas import tpu_sc as plsc

_L = 16


def _wsum(om, a, u, v):
    def term(s, w):
        if isinstance(w, float):
            if w == 0.0:
                return None
            if w == 1.0:
                return s
        return s * w
    t1, t2 = term(om, u), term(a, v)
    if t1 is None and t2 is None:
        return 0.0
    if t1 is None:
        return t2
    if t2 is None:
        return t1
    return t1 + t2


def _make_sc_eval(B, n_coef, C, n_knots):
    NC, NS = 2, 16
    NW = NC * NS
    assert B % NW == 0
    rows_per_w = B // NW
    chunk = 128
    assert rows_per_w % chunk == 0
    n_chunks = rows_per_w // chunk
    groups = chunk // _L
    kmax = n_knots - 5

    mesh = plsc.VectorSubcoreMesh(core_axis_name="c", subcore_axis_name="s",
                                  num_cores=NC, num_subcores=NS)

    @functools.partial(
        pl.kernel,
        out_type=jax.ShapeDtypeStruct((C, B), jnp.float32),
        mesh=mesh,
        scratch_types=[
            pltpu.VMEM((n_knots, chunk), jnp.float32),
            pltpu.VMEM((chunk,), jnp.float32),
            pltpu.VMEM((4, chunk), jnp.float32),
            pltpu.VMEM((chunk,), jnp.int32),
            pltpu.VMEM((n_coef // 2, C, chunk), jnp.float32),
            pltpu.VMEM((n_coef // 2, C, chunk), jnp.float32),
            pltpu.VMEM((C, chunk), jnp.float32),
            pltpu.SemaphoreType.DMA,
            pltpu.SemaphoreType.DMA,
        ],
        compiler_params=pltpu.CompilerParams(needs_layout_passes=False,
                                             use_tc_tiling_on_sc=True),
    )
    def run(knots_hbm, x_hbm, coef_hbm, out_hbm,
            knots_v, x_v, w_v, km3_v, p_a, p_b, out_v, sem_a, sem_b):
        cid = lax.axis_index("c")
        sid = lax.axis_index("s")
        wid = sid * NC + cid
        wbase = wid * rows_per_w
        lane = lax.iota(jnp.int32, _L)
        nh = n_coef // 2

        def slab(c, half):
            base = wbase + c * chunk
            return coef_hbm.at[pl.ds(half * nh, nh), :, pl.ds(base, chunk)]

        pltpu.async_copy(slab(0, 0), p_a, sem_a)

        def chunk_body(c, carry):
            base = wbase + c * chunk
            pltpu.async_copy(slab(c, 1), p_b, sem_b)
            pltpu.sync_copy(knots_hbm.at[:, pl.ds(base, chunk)], knots_v)
            pltpu.sync_copy(x_hbm.at[pl.ds(base, chunk)], x_v)

            def prep_body(g, carry2):
                sl = pl.ds(g * _L, _L)
                x = x_v[sl]
                cnt = jnp.zeros((_L,), jnp.int32)
                for i in range(n_knots):
                    cnt = cnt + jnp.where(knots_v[i, sl] <= x, 1, 0)
                k = jnp.clip(cnt - 1, 3, kmax)
                col = g * _L + lane
                ts = [plsc.load_gather(knots_v, [k - 2 + m, col])
                      for m in range(6)]
                w = [[1.0 if cc == j else 0.0 for cc in range(4)]
                     for j in range(4)]
                for r in range(1, 4):
                    for j in range(3, r - 1, -1):
                        a = (x - ts[j - 1]) / (ts[j + 3 - r] - ts[j - 1])
                        om = 1.0 - a
                        w[j] = [_wsum(om, a, w[j - 1][cc], w[j][cc])
                                for cc in range(4)]
                for j in range(4):
                    w_v[j, sl] = w[3][j]
                km3_v[sl] = k - 3
                return carry2

            lax.fori_loop(0, groups, prep_body, 0)

            def half_sum(p_half, half):
                def sum_body(g, carry2):
                    sl = pl.ds(g * _L, _L)
                    km3 = km3_v[sl]
                    ws = [w_v[j, sl] for j in range(4)]
                    acc = [out_v[ch, sl] for ch in range(C)] \
                        if half else [None] * C
                    for ii in range(nh):
                        i = half * nh + ii
                        di = i - km3
                        wi = None
                        for j in range(max(0, i - (n_coef - 4)),
                                       min(3, i) + 1):
                            t = jnp.where(di == j, ws[j], 0.0)
                            wi = t if wi is None else wi + t
                        for ch in range(C):
                            v = wi * p_half[ii, ch, sl]
                            acc[ch] = v if acc[ch] is None else acc[ch] + v
                    for ch in range(C):
                        out_v[ch, sl] = acc[ch]
                    return carry2
                return sum_body

            pltpu.make_async_copy(slab(c, 0), p_a, sem_a).wait()
            lax.fori_loop(0, groups, half_sum(p_a, 0), 0)
            @pl.when(c + 1 < n_chunks)
            def _():
                pltpu.async_copy(slab(c + 1, 0), p_a, sem_a)
            pltpu.make_async_copy(slab(c, 1), p_b, sem_b).wait()
            lax.fori_loop(0, groups, half_sum(p_b, 1), 0)
            pltpu.sync_copy(out_v, out_hbm.at[:, pl.ds(base, chunk)])
            return carry

        lax.fori_loop(0, n_chunks, chunk_body, 0)

    return run


def kernel(coefficients, knots, inpce):
    B, n_coef, C = coefficients.shape
    n_knots = knots.shape[1]
    coef_t = jnp.transpose(coefficients, (1, 2, 0))
    knots_t = knots.T
    x = inpce.reshape(B)
    run = _make_sc_eval(B, n_coef, C, n_knots)
    out_t = run(knots_t, x, coef_t)
    return out_t.T


# speedup vs baseline: 1.7532x; 1.7532x over previous
import functools

import jax
import jax.numpy as jnp
from jax import lax
from jax.experimental import pallas as pl
from jax.experimental.pallas import tpu as pltpu
from jax.experimental.pallas import tpu_sc as plsc

_L = 16


def _wsum(om, a, u, v):
    def term(s, w):
        if isinstance(w, float):
            if w == 0.0:
                return None
            if w == 1.0:
                return s
        return s * w
    t1, t2 = term(om, u), term(a, v)
    if t1 is None and t2 is None:
        return 0.0
    if t1 is None:
        return t2
    if t2 is None:
        return t1
    return t1 + t2


def _basis_weights(xx, ts):
    w = [[1.0 if cc == j else 0.0 for cc in range(4)] for j in range(4)]
    for r in range(1, 4):
        for j in range(3, r - 1, -1):
            a = (xx - ts[j - 1]) / (ts[j + 3 - r] - ts[j - 1])
            om = 1.0 - a
            w[j] = [_wsum(om, a, w[j - 1][cc], w[j][cc]) for cc in range(4)]
    return w[3]


def _row_weight(i, km3, ws, n_coef):
    di = i - km3
    wi = None
    for j in range(max(0, i - (n_coef - 4)), min(3, i) + 1):
        t = jnp.where(di == j, ws[j], 0.0)
        wi = t if wi is None else wi + t
    return wi


def _make_sc_eval(b_sc, n_coef, C, n_knots):
    NC, NS = 2, 16
    NW = NC * NS
    assert b_sc % NW == 0
    rows_per_w = b_sc // NW
    chunk = 128
    assert rows_per_w % chunk == 0
    n_chunks = rows_per_w // chunk
    groups = chunk // _L
    kmax = n_knots - 5

    mesh = plsc.VectorSubcoreMesh(core_axis_name="c", subcore_axis_name="s",
                                  num_cores=NC, num_subcores=NS)

    @functools.partial(
        pl.kernel,
        out_type=jax.ShapeDtypeStruct((C, b_sc), jnp.float32),
        mesh=mesh,
        scratch_types=[
            pltpu.VMEM((n_knots, chunk), jnp.float32),
            pltpu.VMEM((chunk,), jnp.float32),
            pltpu.VMEM((4, chunk), jnp.float32),
            pltpu.VMEM((chunk,), jnp.int32),
            pltpu.VMEM((n_coef, C, chunk), jnp.float32),
            pltpu.VMEM((C, chunk), jnp.float32),
            pltpu.SemaphoreType.DMA,
        ],
        compiler_params=pltpu.CompilerParams(needs_layout_passes=False,
                                             use_tc_tiling_on_sc=True),
    )
    def run(knots_hbm, x_hbm, coef_hbm, out_hbm,
            knots_v, x_v, w_v, km3_v, p_v, out_v, sem):
        cid = lax.axis_index("c")
        sid = lax.axis_index("s")
        wid = sid * NC + cid
        wbase = wid * rows_per_w
        lane = lax.iota(jnp.int32, _L)

        def chunk_body(c, carry):
            base = wbase + c * chunk
            cpc = pltpu.async_copy(
                coef_hbm.at[:, :, pl.ds(base, chunk)], p_v, sem)
            pltpu.sync_copy(knots_hbm.at[:, pl.ds(base, chunk)], knots_v)
            pltpu.sync_copy(x_hbm.at[pl.ds(base, chunk)], x_v)

            def prep_body(g, carry2):
                sl = pl.ds(g * _L, _L)
                x = x_v[sl]
                cnt = jnp.zeros((_L,), jnp.int32)
                for i in range(n_knots):
                    cnt = cnt + jnp.where(knots_v[i, sl] <= x, 1, 0)
                k = jnp.clip(cnt - 1, 3, kmax)
                col = g * _L + lane
                ts = [plsc.load_gather(knots_v, [k - 2 + m, col])
                      for m in range(6)]
                w = _basis_weights(x, ts)
                for j in range(4):
                    w_v[j, sl] = w[j]
                km3_v[sl] = k - 3
                return carry2

            lax.fori_loop(0, groups, prep_body, 0)
            cpc.wait()

            def sum_body(g, carry2):
                sl = pl.ds(g * _L, _L)
                km3 = km3_v[sl]
                ws = [w_v[j, sl] for j in range(4)]
                acc = [None] * C
                for i in range(n_coef):
                    wi = _row_weight(i, km3, ws, n_coef)
                    for ch in range(C):
                        v = wi * p_v[i, ch, sl]
                        acc[ch] = v if acc[ch] is None else acc[ch] + v
                for ch in range(C):
                    out_v[ch, sl] = acc[ch]
                return carry2

            lax.fori_loop(0, groups, sum_body, 0)
            pltpu.sync_copy(out_v, out_hbm.at[:, pl.ds(base, chunk)])
            return carry

        lax.fori_loop(0, n_chunks, chunk_body, 0)

    return run


def _make_tc_eval(b_off, b_len, n_coef, C, n_knots, bb=512):
    assert b_off % bb == 0 and b_len % bb == 0
    off = b_off // bb
    kmax = n_knots - 5

    def body(knots_ref, x_ref, coef_ref, out_ref):
        t = knots_ref[...]
        xx = x_ref[...]
        cnt = jnp.sum((t <= xx).astype(jnp.int32), axis=0, keepdims=True)
        k = jnp.clip(cnt - 1, 3, kmax)
        iot = lax.broadcasted_iota(jnp.int32, (n_knots, bb), 0)
        ts = [jnp.sum(jnp.where(iot == (k - 2 + m), t, 0.0),
                      axis=0, keepdims=True)
              for m in range(6)]
        ws = _basis_weights(xx, ts)
        km3 = k - 3
        acc = None
        for i in range(n_coef):
            wi = _row_weight(i, km3, ws, n_coef)
            v = wi * coef_ref[i]
            acc = v if acc is None else acc + v
        out_ref[...] = acc

    return pl.pallas_call(
        body,
        grid=(b_len // bb,),
        in_specs=[
            pl.BlockSpec((n_knots, bb), lambda n: (0, off + n)),
            pl.BlockSpec((1, bb), lambda n: (0, off + n)),
            pl.BlockSpec((n_coef, C, bb), lambda n: (0, 0, off + n)),
        ],
        out_specs=pl.BlockSpec((C, bb), lambda n: (0, n)),
        out_shape=jax.ShapeDtypeStruct((C, b_len), jnp.float32),
    )


def kernel(coefficients, knots, inpce):
    B, n_coef, C = coefficients.shape
    n_knots = knots.shape[1]
    coef_t = jnp.transpose(coefficients, (1, 2, 0))
    knots_t = knots.T
    x = inpce.reshape(B)
    b_sc = (B // 4 * 5 // 4) // 4096 * 4096
    sc = _make_sc_eval(b_sc, n_coef, C, n_knots)
    tc = _make_tc_eval(b_sc, B - b_sc, n_coef, C, n_knots)
    out_sc = sc(knots_t, x, coef_t)
    out_tc = tc(knots_t, x.reshape(1, B), coef_t)
    return jnp.concatenate([out_sc, out_tc], axis=1).T


# speedup vs baseline: 2.0045x; 1.1433x over previous
import functools

import jax
import jax.numpy as jnp
from jax import lax
from jax.experimental import pallas as pl
from jax.experimental.pallas import tpu as pltpu
from jax.experimental.pallas import tpu_sc as plsc

_L = 16


def _wsum(om, a, u, v):
    def term(s, w):
        if isinstance(w, float):
            if w == 0.0:
                return None
            if w == 1.0:
                return s
        return s * w
    t1, t2 = term(om, u), term(a, v)
    if t1 is None and t2 is None:
        return 0.0
    if t1 is None:
        return t2
    if t2 is None:
        return t1
    return t1 + t2


def _basis_weights(xx, ts):
    w = [[1.0 if cc == j else 0.0 for cc in range(4)] for j in range(4)]
    for r in range(1, 4):
        for j in range(3, r - 1, -1):
            a = (xx - ts[j - 1]) / (ts[j + 3 - r] - ts[j - 1])
            om = 1.0 - a
            w[j] = [_wsum(om, a, w[j - 1][cc], w[j][cc]) for cc in range(4)]
    return w[3]


def _row_weight(i, km3, ws, n_coef):
    di = i - km3
    wi = None
    for j in range(max(0, i - (n_coef - 4)), min(3, i) + 1):
        t = jnp.where(di == j, ws[j], 0.0)
        wi = t if wi is None else wi + t
    return wi


def _make_sc_eval(b_sc, n_coef, C, n_knots):
    NC, NS = 2, 16
    NW = NC * NS
    assert b_sc % NW == 0
    rows_per_w = b_sc // NW
    chunk = 128
    assert rows_per_w % chunk == 0
    n_chunks = rows_per_w // chunk
    groups = chunk // _L
    kmax = n_knots - 5

    mesh = plsc.VectorSubcoreMesh(core_axis_name="c", subcore_axis_name="s",
                                  num_cores=NC, num_subcores=NS)

    @functools.partial(
        pl.kernel,
        out_type=jax.ShapeDtypeStruct((C, b_sc), jnp.float32),
        mesh=mesh,
        scratch_types=[
            pltpu.VMEM((n_knots, chunk), jnp.float32),
            pltpu.VMEM((chunk,), jnp.float32),
            pltpu.VMEM((4, chunk), jnp.float32),
            pltpu.VMEM((chunk,), jnp.int32),
            pltpu.VMEM((n_coef, C, chunk), jnp.float32),
            pltpu.VMEM((C, chunk), jnp.float32),
            pltpu.SemaphoreType.DMA,
        ],
        compiler_params=pltpu.CompilerParams(needs_layout_passes=False,
                                             use_tc_tiling_on_sc=True),
    )
    def run(knots_hbm, x_hbm, coef_hbm, out_hbm,
            knots_v, x_v, w_v, km3_v, p_v, out_v, sem):
        cid = lax.axis_index("c")
        sid = lax.axis_index("s")
        wid = sid * NC + cid
        wbase = wid * rows_per_w
        lane = lax.iota(jnp.int32, _L)

        def chunk_body(c, carry):
            base = wbase + c * chunk
            cpc = pltpu.async_copy(
                coef_hbm.at[:, :, pl.ds(base, chunk)], p_v, sem)
            pltpu.sync_copy(knots_hbm.at[:, pl.ds(base, chunk)], knots_v)
            pltpu.sync_copy(x_hbm.at[pl.ds(base, chunk)], x_v)

            def prep_body(g, carry2):
                sl = pl.ds(g * _L, _L)
                x = x_v[sl]
                cnt = jnp.zeros((_L,), jnp.int32)
                for i in range(n_knots):
                    cnt = cnt + jnp.where(knots_v[i, sl] <= x, 1, 0)
                k = jnp.clip(cnt - 1, 3, kmax)
                col = g * _L + lane
                ts = [plsc.load_gather(knots_v, [k - 2 + m, col])
                      for m in range(6)]
                w = _basis_weights(x, ts)
                for j in range(4):
                    w_v[j, sl] = w[j]
                km3_v[sl] = k - 3
                return carry2

            lax.fori_loop(0, groups, prep_body, 0)
            cpc.wait()

            def sum_body(g, carry2):
                sl = pl.ds(g * _L, _L)
                km3 = km3_v[sl]
                ws = [w_v[j, sl] for j in range(4)]
                acc = [None] * C
                for i in range(n_coef):
                    wi = _row_weight(i, km3, ws, n_coef)
                    for ch in range(C):
                        v = wi * p_v[i, ch, sl]
                        acc[ch] = v if acc[ch] is None else acc[ch] + v
                for ch in range(C):
                    out_v[ch, sl] = acc[ch]
                return carry2

            lax.fori_loop(0, groups, sum_body, 0)
            pltpu.sync_copy(out_v, out_hbm.at[:, pl.ds(base, chunk)])
            return carry

        lax.fori_loop(0, n_chunks, chunk_body, 0)

    return run


def _make_tc_eval(b_off, b_len, n_coef, C, n_knots, bb=1024):
    assert b_off % bb == 0 and b_len % bb == 0
    off = b_off // bb
    kmax = n_knots - 5

    def body(knots_ref, x_ref, coef_ref, out_ref):
        t = knots_ref[...]
        xx = x_ref[...]
        cnt = jnp.sum((t <= xx).astype(jnp.int32), axis=0, keepdims=True)
        k = jnp.clip(cnt - 1, 3, kmax)
        iot = lax.broadcasted_iota(jnp.int32, (n_knots, bb), 0)
        ts = [jnp.sum(jnp.where(iot == (k - 2 + m), t, 0.0),
                      axis=0, keepdims=True)
              for m in range(6)]
        ws = _basis_weights(xx, ts)
        km3 = k - 3
        acc = None
        for i in range(n_coef):
            wi = _row_weight(i, km3, ws, n_coef)
            v = wi * coef_ref[i]
            acc = v if acc is None else acc + v
        out_ref[...] = acc

    return pl.pallas_call(
        body,
        grid=(b_len // bb,),
        in_specs=[
            pl.BlockSpec((n_knots, bb), lambda n: (0, off + n)),
            pl.BlockSpec((1, bb), lambda n: (0, off + n)),
            pl.BlockSpec((n_coef, C, bb), lambda n: (0, 0, off + n)),
        ],
        out_specs=pl.BlockSpec((C, bb), lambda n: (0, n)),
        out_shape=jax.ShapeDtypeStruct((C, b_len), jnp.float32),
    )


def kernel(coefficients, knots, inpce):
    B, n_coef, C = coefficients.shape
    n_knots = knots.shape[1]
    coef_t = jnp.transpose(coefficients, (1, 2, 0))
    knots_t = knots.T
    x = inpce.reshape(B)
    b_sc = (B * 13 // 32) // 4096 * 4096
    sc = _make_sc_eval(b_sc, n_coef, C, n_knots)
    tc = _make_tc_eval(b_sc, B - b_sc, n_coef, C, n_knots)
    out_sc = sc(knots_t, x, coef_t)
    out_tc = tc(knots_t, x.reshape(1, B), coef_t)
    return jnp.concatenate([out_sc, out_tc], axis=1).T


# speedup vs baseline: 2.1039x; 1.0496x over previous
import functools

import jax
import jax.numpy as jnp
from jax import lax
from jax.experimental import pallas as pl
from jax.experimental.pallas import tpu as pltpu
from jax.experimental.pallas import tpu_sc as plsc

_L = 16


def _wsum(om, a, u, v):
    def term(s, w):
        if isinstance(w, float):
            if w == 0.0:
                return None
            if w == 1.0:
                return s
        return s * w
    t1, t2 = term(om, u), term(a, v)
    if t1 is None and t2 is None:
        return 0.0
    if t1 is None:
        return t2
    if t2 is None:
        return t1
    return t1 + t2


def _basis_weights(xx, ts):
    w = [[1.0 if cc == j else 0.0 for cc in range(4)] for j in range(4)]
    for r in range(1, 4):
        for j in range(3, r - 1, -1):
            a = (xx - ts[j - 1]) / (ts[j + 3 - r] - ts[j - 1])
            om = 1.0 - a
            w[j] = [_wsum(om, a, w[j - 1][cc], w[j][cc]) for cc in range(4)]
    return w[3]


def _row_weight(i, km3, ws, n_coef):
    di = i - km3
    wi = None
    for j in range(max(0, i - (n_coef - 4)), min(3, i) + 1):
        t = jnp.where(di == j, ws[j], 0.0)
        wi = t if wi is None else wi + t
    return wi


def _make_sc_eval(b_sc, n_coef, C, n_knots):
    NC, NS = 2, 16
    NW = NC * NS
    assert b_sc % NW == 0
    rows_per_w = b_sc // NW
    chunk = 128
    assert rows_per_w % chunk == 0
    n_chunks = rows_per_w // chunk
    groups = chunk // _L
    kmax = n_knots - 5

    mesh = plsc.VectorSubcoreMesh(core_axis_name="c", subcore_axis_name="s",
                                  num_cores=NC, num_subcores=NS)

    @functools.partial(
        pl.kernel,
        out_type=jax.ShapeDtypeStruct((C, b_sc), jnp.float32),
        mesh=mesh,
        scratch_types=[
            pltpu.VMEM((n_knots, chunk), jnp.float32),
            pltpu.VMEM((chunk,), jnp.float32),
            pltpu.VMEM((4, chunk), jnp.float32),
            pltpu.VMEM((chunk,), jnp.int32),
            pltpu.VMEM((n_coef, C, chunk), jnp.float32),
            pltpu.VMEM((C, chunk), jnp.float32),
            pltpu.SemaphoreType.DMA,
        ],
        compiler_params=pltpu.CompilerParams(needs_layout_passes=False,
                                             use_tc_tiling_on_sc=True),
    )
    def run(knots_hbm, x_hbm, coef_hbm, out_hbm,
            knots_v, x_v, w_v, km3_v, p_v, out_v, sem):
        cid = lax.axis_index("c")
        sid = lax.axis_index("s")
        wid = sid * NC + cid
        wbase = wid * rows_per_w
        lane = lax.iota(jnp.int32, _L)

        def chunk_body(c, carry):
            base = wbase + c * chunk
            cpc = pltpu.async_copy(
                coef_hbm.at[:, :, pl.ds(base, chunk)], p_v, sem)
            pltpu.sync_copy(knots_hbm.at[:, pl.ds(base, chunk)], knots_v)
            pltpu.sync_copy(x_hbm.at[pl.ds(base, chunk)], x_v)

            def prep_body(g, carry2):
                sl = pl.ds(g * _L, _L)
                x = x_v[sl]
                cnt = jnp.zeros((_L,), jnp.int32)
                for i in range(n_knots):
                    cnt = cnt + jnp.where(knots_v[i, sl] <= x, 1, 0)
                k = jnp.clip(cnt - 1, 3, kmax)
                col = g * _L + lane
                ts = [plsc.load_gather(knots_v, [k - 2 + m, col])
                      for m in range(6)]
                w = _basis_weights(x, ts)
                for j in range(4):
                    w_v[j, sl] = w[j]
                km3_v[sl] = k - 3
                return carry2

            lax.fori_loop(0, groups, prep_body, 0)
            cpc.wait()

            def sum_body(g, carry2):
                sl = pl.ds(g * _L, _L)
                km3 = km3_v[sl]
                ws = [w_v[j, sl] for j in range(4)]
                acc = [None] * C
                for i in range(n_coef):
                    wi = _row_weight(i, km3, ws, n_coef)
                    for ch in range(C):
                        v = wi * p_v[i, ch, sl]
                        acc[ch] = v if acc[ch] is None else acc[ch] + v
                for ch in range(C):
                    out_v[ch, sl] = acc[ch]
                return carry2

            lax.fori_loop(0, groups, sum_body, 0)
            pltpu.sync_copy(out_v, out_hbm.at[:, pl.ds(base, chunk)])
            return carry

        lax.fori_loop(0, n_chunks, chunk_body, 0)

    return run


def _make_tc_eval(b_off, b_len, n_coef, C, n_knots, bb=1024):
    assert b_off % bb == 0 and b_len % bb == 0
    off = b_off // bb
    kmax = n_knots - 5

    def body(knots_ref, x_ref, coef_ref, out_ref):
        t = knots_ref[...]
        xx = x_ref[...]
        cnt = jnp.sum((t <= xx).astype(jnp.int32), axis=0, keepdims=True)
        k = jnp.clip(cnt - 1, 3, kmax)
        iot = lax.broadcasted_iota(jnp.int32, (n_knots, bb), 0)
        ts = [jnp.sum(jnp.where(iot == (k - 2 + m), t, 0.0),
                      axis=0, keepdims=True)
              for m in range(6)]
        ws = _basis_weights(xx, ts)
        km3 = k - 3
        acc = None
        for i in range(n_coef):
            wi = _row_weight(i, km3, ws, n_coef)
            v = wi * coef_ref[i]
            acc = v if acc is None else acc + v
        out_ref[...] = acc

    return pl.pallas_call(
        body,
        grid=(b_len // bb,),
        in_specs=[
            pl.BlockSpec((n_knots, bb), lambda n: (0, off + n)),
            pl.BlockSpec((1, bb), lambda n: (0, off + n)),
            pl.BlockSpec((n_coef, C, bb), lambda n: (0, 0, off + n)),
        ],
        out_specs=pl.BlockSpec((C, bb), lambda n: (0, n)),
        out_shape=jax.ShapeDtypeStruct((C, b_len), jnp.float32),
    )


def kernel(coefficients, knots, inpce):
    B, n_coef, C = coefficients.shape
    n_knots = knots.shape[1]
    coef_t = jnp.transpose(coefficients, (1, 2, 0))
    knots_t = knots.T
    x = inpce.reshape(B)
    b_sc = (B * 3 // 8) // 4096 * 4096
    sc = _make_sc_eval(b_sc, n_coef, C, n_knots)
    tc = _make_tc_eval(b_sc, B - b_sc, n_coef, C, n_knots)
    out_sc = sc(knots_t, x, coef_t)
    out_tc = tc(knots_t, x.reshape(1, B), coef_t)
    return jnp.concatenate([out_sc, out_tc], axis=1).T
